# baseline (speedup 1.0000x reference)
import functools

import jax
import jax.numpy as jnp
from jax import lax
from jax.experimental import pallas as pl
from jax.experimental.pallas import tpu as pltpu
from jax.experimental.pallas import tpu_sc as plsc

_N = 10000
_E = 160000
_D = 256
_H = 256
_EPS = 1e-5

_RB = 400
_NRB = _N // _RB

_NSUB = 16
_B = 128
_NBUF = 3
_NB = 81
_CH = _NB * _B
_EPAD = _NSUB * _CH
_RPT = 640
_ACC_ROWS = 10016
_ZTAIL = _ACC_ROWS - 15 * _RPT
_GARBAGE = 10008


def _mm_stats_body(x_ref, w_ref, h1_ref, stats_ref, acc_ref):
    i = pl.program_id(0)
    h1 = jnp.dot(x_ref[...], w_ref[...], preferred_element_type=jnp.float32)
    h1_ref[...] = h1
    s = jnp.sum(h1, axis=0, keepdims=True)
    s2 = jnp.sum(h1 * h1, axis=0, keepdims=True)
    ss = jnp.concatenate([s, s2], axis=0)

    @pl.when(i == 0)
    def _():
        acc_ref[...] = jnp.zeros_like(acc_ref)

    acc_ref[...] += ss

    @pl.when(i == pl.num_programs(0) - 1)
    def _():
        stats_ref[...] = acc_ref[...]


_mm_stats = pl.pallas_call(
    _mm_stats_body,
    grid=(_NRB,),
    in_specs=[
        pl.BlockSpec((_RB, _D), lambda i: (i, 0)),
        pl.BlockSpec((_D, _H), lambda i: (0, 0)),
    ],
    out_specs=[
        pl.BlockSpec((_RB, _H), lambda i: (i, 0)),
        pl.BlockSpec((2, _H), lambda i: (0, 0)),
    ],
    out_shape=[
        jax.ShapeDtypeStruct((_N, _H), jnp.float32),
        jax.ShapeDtypeStruct((2, _H), jnp.float32),
    ],
    scratch_shapes=[pltpu.VMEM((2, _H), jnp.float32)],
)


def _mm2_body(h1_ref, ab_ref, w_ref, z_ref):
    h1s = h1_ref[...] * ab_ref[0:1, :] + ab_ref[1:2, :]
    z = jnp.dot(h1s, w_ref[...], preferred_element_type=jnp.float32)
    z_ref[0] = z[:, 0:128]
    z_ref[1] = z[:, 128:256]


_mm2 = pl.pallas_call(
    _mm2_body,
    grid=(_NRB,),
    in_specs=[
        pl.BlockSpec((_RB, _H), lambda i: (i, 0)),
        pl.BlockSpec((2, _H), lambda i: (0, 0)),
        pl.BlockSpec((_H, _D), lambda i: (0, 0)),
    ],
    out_specs=pl.BlockSpec((2, _RB, 128), lambda i: (0, i, 0)),
    out_shape=jax.ShapeDtypeStruct((2, _N, 128), jnp.float32),
)


def _stats2_body(h_ref, stats_ref, acc_ref):
    i = pl.program_id(0)
    hb = h_ref[...]
    s = jnp.sum(hb, axis=1)
    s2 = jnp.sum(hb * hb, axis=1)
    ss = jnp.stack([s, s2], axis=0)

    @pl.when(i == 0)
    def _():
        acc_ref[...] = jnp.zeros_like(acc_ref)

    acc_ref[...] += ss

    @pl.when(i == pl.num_programs(0) - 1)
    def _():
        stats_ref[...] = acc_ref[...]


_stats2 = pl.pallas_call(
    _stats2_body,
    grid=(_NRB,),
    in_specs=[pl.BlockSpec((2, _RB, 128), lambda i: (0, i, 0))],
    out_specs=pl.BlockSpec((2, 2, 128), lambda i: (0, 0, 0)),
    out_shape=jax.ShapeDtypeStruct((2, 2, 128), jnp.float32),
    scratch_shapes=[pltpu.VMEM((2, 2, 128), jnp.float32)],
)


def _bn2_body(h_ref, ab_ref, out_ref):
    hb = h_ref[...]
    a = ab_ref[0]
    b = ab_ref[1]
    y = hb * a[:, None, :] + b[:, None, :]
    out_ref[:, 0:128] = y[0]
    out_ref[:, 128:256] = y[1]


_bn2 = pl.pallas_call(
    _bn2_body,
    grid=(_NRB,),
    in_specs=[
        pl.BlockSpec((2, _RB, 128), lambda i: (0, i, 0)),
        pl.BlockSpec((2, 2, 128), lambda i: (0, 0, 0)),
    ],
    out_specs=pl.BlockSpec((_RB, _D), lambda i: (i, 0)),
    out_shape=jax.ShapeDtypeStruct((_N, _D), jnp.float32),
)


def _seg_body(zf_h, src0_h, src1_h, dst_h, zrows_h, out_h,
              sring, dring, rb0, rb1, rb2, acc_s,
              g0, g1, g2, s0, s1, s2, d0, d1, d2, i0, i1, i2):
    c = lax.axis_index("c")
    s = lax.axis_index("s")
    r0 = s * _RPT
    rows = (rb0, rb1, rb2)
    gsem = (g0, g1, g2)
    ssem = (s0, s1, s2)
    dsem = (d0, d1, d2)
    isem = (i0, i1, i2)
    base = s * _CH

    @pl.when(s < _NSUB - 1)
    def _():
        pltpu.sync_copy(zrows_h, acc_s.at[pl.ds(r0, _RPT)])

    @pl.when(s == _NSUB - 1)
    def _():
        pltpu.sync_copy(zrows_h.at[pl.ds(0, _ZTAIL)],
                        acc_s.at[pl.ds(r0, _ZTAIL)])

    def _sidx_start(j, b):
        off = base + j * _B

        @pl.when(c == 0)
        def _():
            pltpu.make_async_copy(src0_h.at[pl.ds(off, _B)], sring.at[b],
                                  isem[b]).start()

        @pl.when(c == 1)
        def _():
            pltpu.make_async_copy(src1_h.at[pl.ds(off, _B)], sring.at[b],
                                  isem[b]).start()

    def _sidx_wait(b):
        pltpu.make_async_copy(src0_h.at[pl.ds(base, _B)], sring.at[b],
                              isem[b]).wait()

    def _didx(j, b):
        return pltpu.make_async_copy(
            dst_h.at[s, pl.ds(j, 1)], dring.at[pl.ds(b, 1)], dsem[b])

    def _gather(b):
        return pltpu.make_async_copy(zf_h.at[sring.at[b]], rows[b], gsem[b])

    def _scatter(b):
        return pltpu.make_async_copy(rows[b], acc_s.at[dring.at[b]], ssem[b])

    for b in range(_NBUF):
        _sidx_start(b, b)
        _didx(b, b).start()
    for b in range(_NBUF - 1):
        _sidx_wait(b)
        _gather(b).start()
    plsc.subcore_barrier()

    def body(g, carry):
        j0 = g * _NBUF
        for b in range(_NBUF):
            j = j0 + b
            bn = (b + _NBUF - 1) % _NBUF
            _gather(b).wait()

            @pl.when(j + _NBUF < _NB)
            def _():
                _sidx_start(j + _NBUF, b)

            _didx(j, b).wait()
            _scatter(b).start(add=True)
            jn = j + _NBUF - 1

            @pl.when((jn < _NB) & (j >= 1))
            def _():
                _scatter(bn).wait()

            @pl.when(jn < _NB)
            def _():
                _didx(jn, bn).start()
                _sidx_wait(bn)
                _gather(bn).start()

        return carry

    lax.fori_loop(0, _NB // _NBUF, body, 0)
    for b in range(_NBUF):
        _scatter(b).wait()
    plsc.subcore_barrier()

    out_base = c * _N + r0

    @pl.when(s < _NSUB - 1)
    def _():
        pltpu.sync_copy(acc_s.at[pl.ds(r0, _RPT)], out_h.at[pl.ds(out_base, _RPT)])

    @pl.when(s == _NSUB - 1)
    def _():
        pltpu.sync_copy(acc_s.at[pl.ds(r0, _N - (_NSUB - 1) * _RPT)],
                        out_h.at[pl.ds(out_base, _N - (_NSUB - 1) * _RPT)])


_seg_sum = functools.partial(
    pl.kernel,
    mesh=plsc.VectorSubcoreMesh(core_axis_name="c", subcore_axis_name="s"),
    out_type=jax.ShapeDtypeStruct((2 * _N, 128), jnp.float32),
    scratch_types=(
        [pltpu.VMEM((_NBUF, _B), jnp.int32), pltpu.VMEM((_NBUF, _B), jnp.int32)]
        + [pltpu.VMEM((_B, 128), jnp.float32) for _ in range(_NBUF)]
        + [pltpu.VMEM_SHARED((_ACC_ROWS, 128), jnp.float32)]
        + [pltpu.SemaphoreType.DMA for _ in range(4 * _NBUF)]
    ),
)(_seg_body)


def kernel(x, edge_index, W1, bn1_gamma, bn1_beta, W2, bn2_gamma, bn2_beta):
    h1, st1 = _mm_stats(x, W1)
    mean1 = st1[0] / _N
    var1 = st1[1] / _N - mean1 * mean1
    a1 = bn1_gamma / jnp.sqrt(var1 + _EPS)
    b1 = bn1_beta - mean1 * a1
    ab1 = jnp.stack([a1, b1])

    z = _mm2(h1, ab1, W2)

    src = edge_index[0]
    dst = edge_index[1]
    pad = _EPAD - _E
    src_p = jnp.concatenate([src, jnp.zeros((pad,), jnp.int32)])
    dst_p = jnp.concatenate([dst, jnp.full((pad,), _GARBAGE, jnp.int32)])
    dst_p = dst_p.reshape(_NSUB, _NB, _B)
    zf = z.reshape(2 * _N, 128)
    zrows = jnp.zeros((_RPT, 128), jnp.float32)

    hf = _seg_sum(zf, src_p, src_p + _N, dst_p, zrows)
    h2 = hf.reshape(2, _N, 128)

    st2 = _stats2(h2)
    mean2 = st2[0] / _N
    var2 = st2[1] / _N - mean2 * mean2
    a2 = bn2_gamma.reshape(2, 128) / jnp.sqrt(var2 + _EPS)
    b2 = bn2_beta.reshape(2, 128) - mean2 * a2
    ab2 = jnp.stack([a2, b2])

    return _bn2(h2, ab2)


# speedup vs baseline: 2.4700x; 2.4700x over previous
import functools

import jax
import jax.numpy as jnp
from jax import lax
from jax.experimental import pallas as pl
from jax.experimental.pallas import tpu as pltpu
from jax.experimental.pallas import tpu_sc as plsc

_N = 10000
_E = 160000
_D = 256
_H = 256
_EPS = 1e-5

_RB = 400
_NRB = _N // _RB

_NSUB = 16
_B = 128
_NBUF = 3
_NB = 81
_CH = _NB * _B
_EPAD = _NSUB * _CH
_BH = _B // 2
_RPT = 640
_ACC_ROWS = 10016
_ZTAIL = _ACC_ROWS - 15 * _RPT


def _mm_stats_body(x_ref, w_ref, h1_ref, stats_ref, acc_ref):
    i = pl.program_id(0)
    h1 = jnp.dot(x_ref[...], w_ref[...], preferred_element_type=jnp.float32)
    h1_ref[...] = h1
    s = jnp.sum(h1, axis=0, keepdims=True)
    s2 = jnp.sum(h1 * h1, axis=0, keepdims=True)
    ss = jnp.concatenate([s, s2], axis=0)

    @pl.when(i == 0)
    def _():
        acc_ref[...] = jnp.zeros_like(acc_ref)

    acc_ref[...] += ss

    @pl.when(i == pl.num_programs(0) - 1)
    def _():
        stats_ref[...] = acc_ref[...]


_mm_stats = pl.pallas_call(
    _mm_stats_body,
    grid=(_NRB,),
    in_specs=[
        pl.BlockSpec((_RB, _D), lambda i: (i, 0)),
        pl.BlockSpec((_D, _H), lambda i: (0, 0)),
    ],
    out_specs=[
        pl.BlockSpec((_RB, _H), lambda i: (i, 0)),
        pl.BlockSpec((2, _H), lambda i: (0, 0)),
    ],
    out_shape=[
        jax.ShapeDtypeStruct((_N, _H), jnp.float32),
        jax.ShapeDtypeStruct((2, _H), jnp.float32),
    ],
    scratch_shapes=[pltpu.VMEM((2, _H), jnp.float32)],
)


def _mm2_body(h1_ref, ab_ref, w_ref, z_ref):
    h1s = h1_ref[...] * ab_ref[0:1, :] + ab_ref[1:2, :]
    z = jnp.dot(h1s, w_ref[...], preferred_element_type=jnp.float32)
    z_ref[0] = z[:, 0:128]
    z_ref[1] = z[:, 128:256]


_mm2 = pl.pallas_call(
    _mm2_body,
    grid=(_NRB,),
    in_specs=[
        pl.BlockSpec((_RB, _H), lambda i: (i, 0)),
        pl.BlockSpec((2, _H), lambda i: (0, 0)),
        pl.BlockSpec((_H, _D), lambda i: (0, 0)),
    ],
    out_specs=pl.BlockSpec((2, _RB, 128), lambda i: (0, i, 0)),
    out_shape=jax.ShapeDtypeStruct((2, _N, 128), jnp.float32),
)


def _stats2_body(h_ref, stats_ref, acc_ref):
    i = pl.program_id(0)
    hb = h_ref[...]
    s = jnp.sum(hb, axis=1)
    s2 = jnp.sum(hb * hb, axis=1)
    ss = jnp.stack([s, s2], axis=0)

    @pl.when(i == 0)
    def _():
        acc_ref[...] = jnp.zeros_like(acc_ref)

    acc_ref[...] += ss

    @pl.when(i == pl.num_programs(0) - 1)
    def _():
        stats_ref[...] = acc_ref[...]


_stats2 = pl.pallas_call(
    _stats2_body,
    grid=(_NRB,),
    in_specs=[pl.BlockSpec((2, _RB, 128), lambda i: (0, i, 0))],
    out_specs=pl.BlockSpec((2, 2, 128), lambda i: (0, 0, 0)),
    out_shape=jax.ShapeDtypeStruct((2, 2, 128), jnp.float32),
    scratch_shapes=[pltpu.VMEM((2, 2, 128), jnp.float32)],
)


def _bn2_body(h_ref, ab_ref, out_ref):
    hb = h_ref[...]
    a = ab_ref[0]
    b = ab_ref[1]
    y = hb * a[:, None, :] + b[:, None, :]
    out_ref[:, 0:128] = y[0]
    out_ref[:, 128:256] = y[1]


_bn2 = pl.pallas_call(
    _bn2_body,
    grid=(_NRB,),
    in_specs=[
        pl.BlockSpec((2, _RB, 128), lambda i: (0, i, 0)),
        pl.BlockSpec((2, 2, 128), lambda i: (0, 0, 0)),
    ],
    out_specs=pl.BlockSpec((_RB, _D), lambda i: (i, 0)),
    out_shape=jax.ShapeDtypeStruct((_N, _D), jnp.float32),
)


def _seg_body(zf_h, src0_h, src1_h, dst_h, zrows_h, out_h,
              sring, dring, *scr):
    c = lax.axis_index("c")
    s = lax.axis_index("s")
    r0 = s * _RPT
    rows = scr[:_NBUF]
    acc_s = scr[_NBUF]
    sems = scr[_NBUF + 1:]
    gsemA = sems[0:_NBUF]
    gsemB = sems[_NBUF:2 * _NBUF]
    ssem = sems[2 * _NBUF:3 * _NBUF]
    dsem = sems[3 * _NBUF:4 * _NBUF]
    isem = sems[4 * _NBUF:5 * _NBUF]
    base = s * _CH

    @pl.when(s < _NSUB - 1)
    def _():
        pltpu.sync_copy(zrows_h, acc_s.at[pl.ds(r0, _RPT)])

    @pl.when(s == _NSUB - 1)
    def _():
        pltpu.sync_copy(zrows_h.at[pl.ds(0, _ZTAIL)],
                        acc_s.at[pl.ds(r0, _ZTAIL)])

    def _sidx_start(j, b):
        off = base + j * _B

        @pl.when(c == 0)
        def _():
            pltpu.make_async_copy(src0_h.at[pl.ds(off, _B)], sring.at[b],
                                  isem[b]).start()

        @pl.when(c == 1)
        def _():
            pltpu.make_async_copy(src1_h.at[pl.ds(off, _B)], sring.at[b],
                                  isem[b]).start()

    def _sidx_wait(b):
        pltpu.make_async_copy(src0_h.at[pl.ds(base, _B)], sring.at[b],
                              isem[b]).wait()

    def _didx(j, b):
        return pltpu.make_async_copy(
            dst_h.at[s, pl.ds(j, 1)], dring.at[pl.ds(b, 1)], dsem[b])

    def _gather_half(b, h, sem):
        return pltpu.make_async_copy(
            zf_h.at[sring.at[b, pl.ds(h * _BH, _BH)]],
            rows[b].at[pl.ds(h * _BH, _BH)], sem)

    def _gather_start(b):
        _gather_half(b, 0, gsemA[b]).start()
        _gather_half(b, 1, gsemB[b]).start()

    def _gather_wait(b):
        _gather_half(b, 0, gsemA[b]).wait()
        _gather_half(b, 1, gsemB[b]).wait()

    def _scatter(b):
        return pltpu.make_async_copy(rows[b], acc_s.at[dring.at[b]], ssem[b])

    for b in range(_NBUF):
        _sidx_start(b, b)
        _didx(b, b).start()
    for b in range(_NBUF - 1):
        _sidx_wait(b)
        _gather_start(b)
    plsc.subcore_barrier()

    def body(g, carry):
        j0 = g * _NBUF
        for b in range(_NBUF):
            j = j0 + b
            bn = (b + _NBUF - 1) % _NBUF
            _gather_wait(b)

            @pl.when(j + _NBUF < _NB)
            def _():
                _sidx_start(j + _NBUF, b)

            _didx(j, b).wait()
            _scatter(b).start(add=True)
            jn = j + _NBUF - 1

            @pl.when((jn < _NB) & (j >= 1))
            def _():
                _scatter(bn).wait()

            @pl.when((jn < _NB) & (jn >= _NBUF))
            def _():
                _didx(jn, bn).start()

            @pl.when(jn < _NB)
            def _():
                _sidx_wait(bn)
                _gather_start(bn)

        return carry

    lax.fori_loop(0, _NB // _NBUF, body, 0)
    for b in range(_NBUF):
        _scatter(b).wait()
    plsc.subcore_barrier()

    out_base = c * _N + r0

    @pl.when(s < _NSUB - 1)
    def _():
        pltpu.sync_copy(acc_s.at[pl.ds(r0, _RPT)], out_h.at[pl.ds(out_base, _RPT)])

    @pl.when(s == _NSUB - 1)
    def _():
        pltpu.sync_copy(acc_s.at[pl.ds(r0, _N - (_NSUB - 1) * _RPT)],
                        out_h.at[pl.ds(out_base, _N - (_NSUB - 1) * _RPT)])


_seg_sum = functools.partial(
    pl.kernel,
    mesh=plsc.VectorSubcoreMesh(core_axis_name="c", subcore_axis_name="s"),
    out_type=jax.ShapeDtypeStruct((2 * _N, 128), jnp.float32),
    scratch_types=(
        [pltpu.VMEM((_NBUF, _B), jnp.int32), pltpu.VMEM((_NBUF, _B), jnp.int32)]
        + [pltpu.VMEM((_B, 128), jnp.float32) for _ in range(_NBUF)]
        + [pltpu.VMEM_SHARED((_ACC_ROWS, 128), jnp.float32)]
        + [pltpu.SemaphoreType.DMA for _ in range(5 * _NBUF)]
    ),
)(_seg_body)


def kernel(x, edge_index, W1, bn1_gamma, bn1_beta, W2, bn2_gamma, bn2_beta):
    h1, st1 = _mm_stats(x, W1)
    mean1 = st1[0] / _N
    var1 = st1[1] / _N - mean1 * mean1
    a1 = bn1_gamma / jnp.sqrt(var1 + _EPS)
    b1 = bn1_beta - mean1 * a1
    ab1 = jnp.stack([a1, b1])

    z = _mm2(h1, ab1, W2)

    src = edge_index[0]
    dst = edge_index[1]
    pad = _EPAD - _E
    pad_src = jnp.arange(pad, dtype=jnp.int32) % _N
    pad_dst = _N + (jnp.arange(pad, dtype=jnp.int32) % (_ACC_ROWS - _N))
    src_p = jnp.concatenate([src, pad_src])
    dst_p = jnp.concatenate([dst, pad_dst]).reshape(_NSUB, _NB, _B)
    zf = z.reshape(2 * _N, 128)
    zrows = jnp.zeros((_RPT, 128), jnp.float32)

    hf = _seg_sum(zf, src_p, src_p + _N, dst_p, zrows)
    h2 = hf.reshape(2, _N, 128)

    st2 = _stats2(h2)
    mean2 = st2[0] / _N
    var2 = st2[1] / _N - mean2 * mean2
    a2 = bn2_gamma.reshape(2, 128) / jnp.sqrt(var2 + _EPS)
    b2 = bn2_beta.reshape(2, 128) - mean2 * a2
    ab2 = jnp.stack([a2, b2])

    return _bn2(h2, ab2)
